# Initial kernel scaffold; baseline (speedup 1.0000x reference)
import jax
import jax.numpy as jnp
from jax.experimental import pallas as pl


def kernel(x, edge_attr, externals, params, edge_index, batch):
    raise NotImplementedError("write your pallas kernel here")


# speedup vs baseline: 1.4778x; 1.4778x over previous
import functools

import jax
import jax.numpy as jnp
from jax import lax
from jax.experimental import pallas as pl
from jax.experimental.pallas import tpu as pltpu
from jax.experimental.pallas import tpu_sc as plsc

N = 10000
E = 160000
G = 64
N_PAD = 10240
E_PAD = 163840
BLK = 512
EBLK = 2048
NBLKS = N_PAD // BLK
EBLKS = E_PAD // EBLK
CHUNK = 128
NCHUNKS = E_PAD // (32 * CHUNK)
HC = 128
ZROWS = 64
F32 = jnp.float32


def _prep_body(we, be, wp0, bp0, wp1, bp1, a1, c1, a2, c2):
    a1_v = jnp.dot(we[...], wp0[...], preferred_element_type=F32)
    c1_v = jnp.dot(be[...], wp0[...], preferred_element_type=F32) + bp0[...]
    a1[...] = a1_v
    c1[...] = c1_v
    a2[...] = jnp.dot(a1_v, wp1[...], preferred_element_type=F32)
    c2[...] = jnp.dot(c1_v, wp1[...], preferred_element_type=F32) + bp1[...]


def _prep(we, be, wp0, bp0, wp1, bp1):
    return pl.pallas_call(
        _prep_body,
        out_shape=[
            jax.ShapeDtypeStruct((16, 128), F32),
            jax.ShapeDtypeStruct((1, 128), F32),
            jax.ShapeDtypeStruct((16, 64), F32),
            jax.ShapeDtypeStruct((1, 64), F32),
        ],
    )(we, be, wp0, bp0, wp1, bp1)


def _mm_body(x_ref, w_ref, b_ref, o_ref):
    o_ref[...] = (
        jnp.dot(x_ref[...], w_ref[...], preferred_element_type=F32) + b_ref[...]
    )


def _matmul_rows(x, w, b, blk):
    rows, fin = x.shape
    fout = w.shape[1]
    grid = rows // blk
    return pl.pallas_call(
        _mm_body,
        grid=(grid,),
        in_specs=[
            pl.BlockSpec((blk, fin), lambda i: (i, 0)),
            pl.BlockSpec((fin, fout), lambda i: (0, 0)),
            pl.BlockSpec((1, fout), lambda i: (0, 0)),
        ],
        out_specs=pl.BlockSpec((blk, fout), lambda i: (i, 0)),
        out_shape=jax.ShapeDtypeStruct((rows, fout), F32),
    )(x, w, b)


def _sc_msg_body(x_hbm, e_hbm, src_hbm, dst_hbm, out_hbm,
                 src_v, dst_v, xg_v, e_v, zb_v, acc_sh, sem):
    c = lax.axis_index("c")
    s = lax.axis_index("s")
    nj = HC // 16
    zero16 = jnp.zeros((16,), F32)

    def zrow(i, _):
        for j in range(nj):
            zb_v[i, pl.ds(16 * j, 16)] = zero16
        return 0
    lax.fori_loop(0, ZROWS, zrow, 0)
    stripe = 640 * s

    def zcopy(t, _):
        pltpu.sync_copy(zb_v, acc_sh.at[pl.ds(stripe + t * ZROWS, ZROWS)])
        return 0
    lax.fori_loop(0, 640 // ZROWS, zcopy, 0)
    plsc.subcore_barrier()

    wbase = (c * 16 + s) * (NCHUNKS * CHUNK)

    def chunk(k, _):
        base = wbase + k * CHUNK
        pltpu.sync_copy(src_hbm.at[pl.ds(base, CHUNK)], src_v)
        pltpu.sync_copy(dst_hbm.at[pl.ds(base, CHUNK)], dst_v)
        pltpu.async_copy(x_hbm.at[src_v], xg_v, sem).wait()
        pltpu.sync_copy(e_hbm.at[pl.ds(base, CHUNK)], e_v)

        def row(i, _):
            for j in range(nj):
                v = xg_v[i, pl.ds(16 * j, 16)] + e_v[i, pl.ds(16 * j, 16)]
                xg_v[i, pl.ds(16 * j, 16)] = jnp.maximum(v, 0.0)
            return 0
        lax.fori_loop(0, CHUNK, row, 0)
        pltpu.sync_copy(xg_v, acc_sh.at[dst_v], add=True)
        return 0

    lax.fori_loop(0, NCHUNKS, chunk, 0)
    plsc.subcore_barrier()
    pltpu.sync_copy(acc_sh.at[pl.ds(640 * s, 640)],
                    out_hbm.at[c, pl.ds(640 * s, 640)])


@jax.jit
def _sc_msg(x_part, e_part, src, dst):
    mesh = plsc.VectorSubcoreMesh(core_axis_name="c", subcore_axis_name="s")
    return pl.kernel(
        _sc_msg_body,
        out_type=jax.ShapeDtypeStruct((2, N_PAD, HC), F32),
        mesh=mesh,
        scratch_types=[
            pltpu.VMEM((CHUNK,), jnp.int32),
            pltpu.VMEM((CHUNK,), jnp.int32),
            pltpu.VMEM((CHUNK, HC), F32),
            pltpu.VMEM((CHUNK, HC), F32),
            pltpu.VMEM((ZROWS, HC), F32),
            pltpu.VMEM_SHARED((N_PAD, HC), F32),
            pltpu.SemaphoreType.DMA,
        ],
    )(x_part, e_part, src, dst)


def _layerA_body(x_ref, p_ref, w1, b1, w2, b2, y_ref, st_ref):
    i = pl.program_id(0)
    h = x_ref.shape[1]
    h1 = x_ref[...] + p_ref[0] + p_ref[1]
    t = jnp.maximum(
        jnp.dot(h1, w1[...], preferred_element_type=F32) + b1[...], 0.0)
    y = jnp.dot(t, w2[...], preferred_element_type=F32) + b2[...]
    y_ref[...] = y
    row = i * BLK + lax.broadcasted_iota(jnp.int32, (BLK, 1), 0)
    m = jnp.where(row < N, 1.0, 0.0).astype(F32)
    ym = y * m
    s0 = jnp.sum(ym, axis=0, keepdims=True)
    s1 = jnp.sum(ym * y, axis=0, keepdims=True)
    upd = jnp.concatenate([s0, s1, jnp.zeros((6, h), F32)], axis=0)

    @pl.when(i == 0)
    def _init():
        st_ref[...] = jnp.zeros_like(st_ref)

    st_ref[...] += upd


def _layerA(x, parts, w1, b1, w2, b2):
    h = x.shape[1]
    return pl.pallas_call(
        _layerA_body,
        grid=(NBLKS,),
        in_specs=[
            pl.BlockSpec((BLK, h), lambda i: (i, 0)),
            pl.BlockSpec((2, BLK, h), lambda i: (0, i, 0)),
            pl.BlockSpec((h, h), lambda i: (0, 0)),
            pl.BlockSpec((1, h), lambda i: (0, 0)),
            pl.BlockSpec((h, h), lambda i: (0, 0)),
            pl.BlockSpec((1, h), lambda i: (0, 0)),
        ],
        out_specs=[
            pl.BlockSpec((BLK, h), lambda i: (i, 0)),
            pl.BlockSpec((8, h), lambda i: (0, 0)),
        ],
        out_shape=[
            jax.ShapeDtypeStruct((N_PAD, h), F32),
            jax.ShapeDtypeStruct((8, h), F32),
        ],
    )(x, parts, w1, b1, w2, b2)


def _layerB_body(proj, y_ref, st_ref, g_ref, be_ref, *rest):
    if proj:
        wp, bp, o_ref = rest
    else:
        (o_ref,) = rest
    mean = st_ref[0:1, :] * (1.0 / N)
    var = st_ref[1:2, :] * (1.0 / N) - mean * mean
    inv = lax.rsqrt(var + 1e-5)
    xn = jnp.maximum((y_ref[...] - mean) * inv * g_ref[...] + be_ref[...], 0.0)
    if proj:
        o_ref[...] = (
            jnp.dot(xn, wp[...], preferred_element_type=F32) + bp[...])
    else:
        o_ref[...] = xn


def _layerB(y, st, gamma, beta, wp=None, bp=None):
    h = y.shape[1]
    proj = wp is not None
    h2 = wp.shape[1] if proj else h
    in_specs = [
        pl.BlockSpec((BLK, h), lambda i: (i, 0)),
        pl.BlockSpec((8, h), lambda i: (0, 0)),
        pl.BlockSpec((1, h), lambda i: (0, 0)),
        pl.BlockSpec((1, h), lambda i: (0, 0)),
    ]
    args = [y, st, gamma, beta]
    if proj:
        in_specs += [
            pl.BlockSpec((h, h2), lambda i: (0, 0)),
            pl.BlockSpec((1, h2), lambda i: (0, 0)),
        ]
        args += [wp, bp]
    return pl.pallas_call(
        functools.partial(_layerB_body, proj),
        grid=(NBLKS,),
        in_specs=in_specs,
        out_specs=pl.BlockSpec((BLK, h2), lambda i: (i, 0)),
        out_shape=jax.ShapeDtypeStruct((N_PAD, h2), F32),
    )(*args)


def _pool_body(x_ref, b_ref, p_ref, c_ref):
    i = pl.program_id(0)
    b = b_ref[0, 0, :]
    oh = jnp.where(
        b[:, None] == lax.broadcasted_iota(jnp.int32, (BLK, G), 1), 1.0, 0.0
    ).astype(F32)
    psum = lax.dot_general(oh, x_ref[...], (((0,), (0,)), ((), ())),
                           preferred_element_type=F32)
    cnt = jnp.sum(oh, axis=0, keepdims=True)
    cupd = jnp.concatenate([cnt, jnp.zeros((7, G), F32)], axis=0)

    @pl.when(i == 0)
    def _init():
        p_ref[...] = jnp.zeros_like(p_ref)
        c_ref[...] = jnp.zeros_like(c_ref)

    p_ref[...] += psum
    c_ref[...] += cupd


def _pool(x, batch_r):
    h = x.shape[1]
    return pl.pallas_call(
        _pool_body,
        grid=(NBLKS,),
        in_specs=[
            pl.BlockSpec((BLK, h), lambda i: (i, 0)),
            pl.BlockSpec((1, 1, BLK), lambda i: (i, 0, 0)),
        ],
        out_specs=[
            pl.BlockSpec((G, h), lambda i: (0, 0)),
            pl.BlockSpec((8, G), lambda i: (0, 0)),
        ],
        out_shape=[
            jax.ShapeDtypeStruct((G, h), F32),
            jax.ShapeDtypeStruct((8, G), F32),
        ],
    )(x, batch_r)


def _head_body(p_ref, c_ref, ex_ref, ew1, eb1, ew2, eb2, hw1, hb1, hw2r, hb2,
               o_ref):
    ge = p_ref[...] / jnp.maximum(c_ref[0:1, :], 1.0).reshape(G, 1)
    ee = jnp.maximum(
        jnp.dot(ex_ref[...], ew1[...], preferred_element_type=F32) + eb1[...],
        0.0)
    ee = jnp.dot(ee, ew2[...], preferred_element_type=F32) + eb2[...]
    comb = jnp.concatenate([ge, ee], axis=1)
    hh = jnp.maximum(
        jnp.dot(comb, hw1[...], preferred_element_type=F32) + hb1[...], 0.0)
    o = jnp.sum(hh * hw2r[...], axis=1, keepdims=True).reshape(1, G)
    o_ref[...] = jnp.concatenate(
        [o + hb2[...], jnp.zeros((7, G), F32)], axis=0)


def _head(pooled, cnts, ex, ew1, eb1, ew2, eb2, hw1, hb1, hw2r, hb2):
    return pl.pallas_call(
        _head_body,
        out_shape=jax.ShapeDtypeStruct((8, G), F32),
    )(pooled, cnts, ex, ew1, eb1, ew2, eb2, hw1, hb1, hw2r, hb2)


def _r1(v):
    return v.reshape(1, -1)


@jax.jit
def _run(x, edge_attr, externals, params, edge_index, batch):
    xp = jnp.zeros((N_PAD, 256), F32).at[:N].set(x)
    ea = jnp.zeros((E_PAD, 16), F32).at[:E].set(edge_attr)
    src = jnp.zeros((E_PAD,), jnp.int32).at[:E].set(edge_index[0].astype(jnp.int32))
    dst = jnp.full((E_PAD,), N_PAD - 1, jnp.int32).at[:E].set(
        edge_index[1].astype(jnp.int32))
    batch_p = jnp.full((N_PAD,), G, jnp.int32).at[:N].set(batch.astype(jnp.int32))
    batch_r = batch_p.reshape(NBLKS, 1, BLK)

    we, be = params['edge_enc']
    wp0e, bp0e = params['eprojs'][0]
    wp1e, bp1e = params['eprojs'][1]
    a1, c1, a2, c2 = _prep(we, _r1(be), wp0e, _r1(bp0e), wp1e, _r1(bp1e))

    wn, bn = params['node_enc']
    h = _matmul_rows(xp, wn, _r1(bn), BLK)

    edge_mats = [(we, _r1(be)), (a1, c1), (a2, c2)]
    widths = [256, 128, 64]
    for i in range(3):
        hw = widths[i]
        A, cvec = edge_mats[i]
        if hw > HC:
            plist = []
            for j in range(hw // HC):
                e_j = _matmul_rows(ea, A[:, j * HC:(j + 1) * HC],
                                   cvec[:, j * HC:(j + 1) * HC], EBLK)
                plist.append(_sc_msg(h[:, j * HC:(j + 1) * HC], e_j, src, dst))
            parts = jnp.concatenate(plist, axis=2)
        elif hw == HC:
            e_i = _matmul_rows(ea, A, cvec, EBLK)
            parts = _sc_msg(h, e_i, src, dst)
        else:
            ap = jnp.zeros((16, HC), F32).at[:, :hw].set(A)
            cp = jnp.zeros((1, HC), F32).at[:, :hw].set(cvec)
            e_i = _matmul_rows(ea, ap, cp, EBLK)
            xpad = jnp.zeros((N_PAD, HC), F32).at[:, :hw].set(h)
            parts = _sc_msg(xpad, e_i, src, dst)[:, :, :hw]
        w1, b1, w2, b2 = params['convs'][i]
        y, st = _layerA(h, parts, w1, _r1(b1), w2, _r1(b2))
        gamma, beta = params['bns'][i]
        if i < 2:
            wp, bp = params['projs'][i]
            h = _layerB(y, st, _r1(gamma), _r1(beta), wp, _r1(bp))
        else:
            h = _layerB(y, st, _r1(gamma), _r1(beta))

    pooled, cnts = _pool(h, batch_r)
    ew1, eb1, ew2, eb2 = params['ext']
    hw1, hb1, hw2, hb2 = params['head']
    out = _head(pooled, cnts, externals,
                ew1, _r1(eb1), ew2, _r1(eb2),
                hw1, _r1(hb1), _r1(hw2[:, 0]),
                jnp.broadcast_to(hb2.reshape(1, 1), (1, G)))
    return out[0]


def kernel(x, edge_attr, externals, params, edge_index, batch):
    return _run(x, edge_attr, externals, params, edge_index, batch)


# speedup vs baseline: 1.6593x; 1.1228x over previous
import functools

import jax
import jax.numpy as jnp
from jax import lax
from jax.experimental import pallas as pl
from jax.experimental.pallas import tpu as pltpu
from jax.experimental.pallas import tpu_sc as plsc

N = 10000
E = 160000
G = 64
N_PAD = 10240
E_PAD = 163840
BLK = 512
EBLK = 2048
NBLKS = N_PAD // BLK
EBLKS = E_PAD // EBLK
CHUNK = 128
NCHUNKS = E_PAD // (32 * CHUNK)
HC = 128
ZROWS = 64
F32 = jnp.float32


def _prep_body(we, be, wp0, bp0, wp1, bp1, a1, c1, a2, c2):
    a1_v = jnp.dot(we[...], wp0[...], preferred_element_type=F32)
    c1_v = jnp.dot(be[...], wp0[...], preferred_element_type=F32) + bp0[...]
    a1[...] = a1_v
    c1[...] = c1_v
    a2[...] = jnp.dot(a1_v, wp1[...], preferred_element_type=F32)
    c2[...] = jnp.dot(c1_v, wp1[...], preferred_element_type=F32) + bp1[...]


def _prep(we, be, wp0, bp0, wp1, bp1):
    return pl.pallas_call(
        _prep_body,
        out_shape=[
            jax.ShapeDtypeStruct((16, 128), F32),
            jax.ShapeDtypeStruct((1, 128), F32),
            jax.ShapeDtypeStruct((16, 64), F32),
            jax.ShapeDtypeStruct((1, 64), F32),
        ],
    )(we, be, wp0, bp0, wp1, bp1)


def _mm_body(x_ref, w_ref, b_ref, o_ref):
    o_ref[...] = (
        jnp.dot(x_ref[...], w_ref[...], preferred_element_type=F32) + b_ref[...]
    )


def _matmul_rows(x, w, b, blk):
    rows, fin = x.shape
    fout = w.shape[1]
    grid = rows // blk
    return pl.pallas_call(
        _mm_body,
        grid=(grid,),
        in_specs=[
            pl.BlockSpec((blk, fin), lambda i: (i, 0)),
            pl.BlockSpec((fin, fout), lambda i: (0, 0)),
            pl.BlockSpec((1, fout), lambda i: (0, 0)),
        ],
        out_specs=pl.BlockSpec((blk, fout), lambda i: (i, 0)),
        out_shape=jax.ShapeDtypeStruct((rows, fout), F32),
    )(x, w, b)


def _sc_msg_body(x_hbm, e_hbm, src_hbm, dst_hbm, out_hbm,
                 src_v, dst_v, xg_v, zb_v, acc_sh, sem):
    c = lax.axis_index("c")
    s = lax.axis_index("s")
    nj = HC // 16
    zero16 = jnp.zeros((16,), F32)

    def zrow(i, _):
        for j in range(nj):
            zb_v[i, pl.ds(16 * j, 16)] = zero16
        return 0
    lax.fori_loop(0, ZROWS, zrow, 0)
    stripe = 640 * s

    def zcopy(t, _):
        pltpu.sync_copy(zb_v, acc_sh.at[pl.ds(stripe + t * ZROWS, ZROWS)])
        return 0
    lax.fori_loop(0, 640 // ZROWS, zcopy, 0)
    plsc.subcore_barrier()

    wrow = (c * 16 + s) * NCHUNKS
    pltpu.sync_copy(src_hbm.at[pl.ds(wrow, NCHUNKS)], src_v)
    pltpu.sync_copy(dst_hbm.at[pl.ds(wrow, NCHUNKS)], dst_v)

    def chunk(k, _):
        pltpu.sync_copy(e_hbm.at[pl.ds((wrow + k) * CHUNK, CHUNK)], xg_v)
        pltpu.async_copy(x_hbm.at[src_v.at[k]], xg_v, sem, add=True).wait()

        def row(i, _):
            for j in range(nj):
                xg_v[i, pl.ds(16 * j, 16)] = jnp.maximum(
                    xg_v[i, pl.ds(16 * j, 16)], 0.0)
            return 0
        lax.fori_loop(0, CHUNK, row, 0)
        pltpu.sync_copy(xg_v, acc_sh.at[dst_v.at[k]], add=True)
        return 0

    lax.fori_loop(0, NCHUNKS, chunk, 0)
    plsc.subcore_barrier()
    pltpu.sync_copy(acc_sh.at[pl.ds(640 * s, 640)],
                    out_hbm.at[c, pl.ds(640 * s, 640)])


@jax.jit
def _sc_msg(x_part, e_part, src_r, dst_r):
    mesh = plsc.VectorSubcoreMesh(core_axis_name="c", subcore_axis_name="s")
    return pl.kernel(
        _sc_msg_body,
        out_type=jax.ShapeDtypeStruct((2, N_PAD, HC), F32),
        mesh=mesh,
        scratch_types=[
            pltpu.VMEM((NCHUNKS, CHUNK), jnp.int32),
            pltpu.VMEM((NCHUNKS, CHUNK), jnp.int32),
            pltpu.VMEM((CHUNK, HC), F32),
            pltpu.VMEM((ZROWS, HC), F32),
            pltpu.VMEM_SHARED((N_PAD, HC), F32),
            pltpu.SemaphoreType.DMA,
        ],
    )(x_part, e_part, src_r, dst_r)


def _layerA_body(x_ref, p_ref, w1, b1, w2, b2, y_ref, st_ref):
    i = pl.program_id(0)
    h = x_ref.shape[1]
    h1 = x_ref[...] + p_ref[0] + p_ref[1]
    t = jnp.maximum(
        jnp.dot(h1, w1[...], preferred_element_type=F32) + b1[...], 0.0)
    y = jnp.dot(t, w2[...], preferred_element_type=F32) + b2[...]
    y_ref[...] = y
    row = i * BLK + lax.broadcasted_iota(jnp.int32, (BLK, 1), 0)
    m = jnp.where(row < N, 1.0, 0.0).astype(F32)
    ym = y * m
    s0 = jnp.sum(ym, axis=0, keepdims=True)
    s1 = jnp.sum(ym * y, axis=0, keepdims=True)
    upd = jnp.concatenate([s0, s1, jnp.zeros((6, h), F32)], axis=0)

    @pl.when(i == 0)
    def _init():
        st_ref[...] = jnp.zeros_like(st_ref)

    st_ref[...] += upd


def _layerA(x, parts, w1, b1, w2, b2):
    h = x.shape[1]
    return pl.pallas_call(
        _layerA_body,
        grid=(NBLKS,),
        in_specs=[
            pl.BlockSpec((BLK, h), lambda i: (i, 0)),
            pl.BlockSpec((2, BLK, h), lambda i: (0, i, 0)),
            pl.BlockSpec((h, h), lambda i: (0, 0)),
            pl.BlockSpec((1, h), lambda i: (0, 0)),
            pl.BlockSpec((h, h), lambda i: (0, 0)),
            pl.BlockSpec((1, h), lambda i: (0, 0)),
        ],
        out_specs=[
            pl.BlockSpec((BLK, h), lambda i: (i, 0)),
            pl.BlockSpec((8, h), lambda i: (0, 0)),
        ],
        out_shape=[
            jax.ShapeDtypeStruct((N_PAD, h), F32),
            jax.ShapeDtypeStruct((8, h), F32),
        ],
    )(x, parts, w1, b1, w2, b2)


def _layerB_body(proj, y_ref, st_ref, g_ref, be_ref, *rest):
    if proj:
        wp, bp, o_ref = rest
    else:
        (o_ref,) = rest
    mean = st_ref[0:1, :] * (1.0 / N)
    var = st_ref[1:2, :] * (1.0 / N) - mean * mean
    inv = lax.rsqrt(var + 1e-5)
    xn = jnp.maximum((y_ref[...] - mean) * inv * g_ref[...] + be_ref[...], 0.0)
    if proj:
        o_ref[...] = (
            jnp.dot(xn, wp[...], preferred_element_type=F32) + bp[...])
    else:
        o_ref[...] = xn


def _layerB(y, st, gamma, beta, wp=None, bp=None):
    h = y.shape[1]
    proj = wp is not None
    h2 = wp.shape[1] if proj else h
    in_specs = [
        pl.BlockSpec((BLK, h), lambda i: (i, 0)),
        pl.BlockSpec((8, h), lambda i: (0, 0)),
        pl.BlockSpec((1, h), lambda i: (0, 0)),
        pl.BlockSpec((1, h), lambda i: (0, 0)),
    ]
    args = [y, st, gamma, beta]
    if proj:
        in_specs += [
            pl.BlockSpec((h, h2), lambda i: (0, 0)),
            pl.BlockSpec((1, h2), lambda i: (0, 0)),
        ]
        args += [wp, bp]
    return pl.pallas_call(
        functools.partial(_layerB_body, proj),
        grid=(NBLKS,),
        in_specs=in_specs,
        out_specs=pl.BlockSpec((BLK, h2), lambda i: (i, 0)),
        out_shape=jax.ShapeDtypeStruct((N_PAD, h2), F32),
    )(*args)


def _pool_body(x_ref, b_ref, p_ref, c_ref):
    i = pl.program_id(0)
    b = b_ref[0, 0, :]
    oh = jnp.where(
        b[:, None] == lax.broadcasted_iota(jnp.int32, (BLK, G), 1), 1.0, 0.0
    ).astype(F32)
    psum = lax.dot_general(oh, x_ref[...], (((0,), (0,)), ((), ())),
                           preferred_element_type=F32)
    cnt = jnp.sum(oh, axis=0, keepdims=True)
    cupd = jnp.concatenate([cnt, jnp.zeros((7, G), F32)], axis=0)

    @pl.when(i == 0)
    def _init():
        p_ref[...] = jnp.zeros_like(p_ref)
        c_ref[...] = jnp.zeros_like(c_ref)

    p_ref[...] += psum
    c_ref[...] += cupd


def _pool(x, batch_r):
    h = x.shape[1]
    return pl.pallas_call(
        _pool_body,
        grid=(NBLKS,),
        in_specs=[
            pl.BlockSpec((BLK, h), lambda i: (i, 0)),
            pl.BlockSpec((1, 1, BLK), lambda i: (i, 0, 0)),
        ],
        out_specs=[
            pl.BlockSpec((G, h), lambda i: (0, 0)),
            pl.BlockSpec((8, G), lambda i: (0, 0)),
        ],
        out_shape=[
            jax.ShapeDtypeStruct((G, h), F32),
            jax.ShapeDtypeStruct((8, G), F32),
        ],
    )(x, batch_r)


def _head_body(p_ref, c_ref, ex_ref, ew1, eb1, ew2, eb2, hw1, hb1, hw2r, hb2,
               o_ref):
    ge = p_ref[...] / jnp.maximum(c_ref[0:1, :], 1.0).reshape(G, 1)
    ee = jnp.maximum(
        jnp.dot(ex_ref[...], ew1[...], preferred_element_type=F32) + eb1[...],
        0.0)
    ee = jnp.dot(ee, ew2[...], preferred_element_type=F32) + eb2[...]
    comb = jnp.concatenate([ge, ee], axis=1)
    hh = jnp.maximum(
        jnp.dot(comb, hw1[...], preferred_element_type=F32) + hb1[...], 0.0)
    o = jnp.sum(hh * hw2r[...], axis=1, keepdims=True).reshape(1, G)
    o_ref[...] = jnp.concatenate(
        [o + hb2[...], jnp.zeros((7, G), F32)], axis=0)


def _head(pooled, cnts, ex, ew1, eb1, ew2, eb2, hw1, hb1, hw2r, hb2):
    return pl.pallas_call(
        _head_body,
        out_shape=jax.ShapeDtypeStruct((8, G), F32),
    )(pooled, cnts, ex, ew1, eb1, ew2, eb2, hw1, hb1, hw2r, hb2)


def _r1(v):
    return v.reshape(1, -1)


@jax.jit
def _run(x, edge_attr, externals, params, edge_index, batch):
    xp = jnp.zeros((N_PAD, 256), F32).at[:N].set(x)
    ea = jnp.zeros((E_PAD, 16), F32).at[:E].set(edge_attr)
    src = jnp.zeros((E_PAD,), jnp.int32).at[:E].set(edge_index[0].astype(jnp.int32))
    dst = jnp.full((E_PAD,), N_PAD - 1, jnp.int32).at[:E].set(
        edge_index[1].astype(jnp.int32))
    src = src.reshape(E_PAD // CHUNK, CHUNK)
    dst = dst.reshape(E_PAD // CHUNK, CHUNK)
    batch_p = jnp.full((N_PAD,), G, jnp.int32).at[:N].set(batch.astype(jnp.int32))
    batch_r = batch_p.reshape(NBLKS, 1, BLK)

    we, be = params['edge_enc']
    wp0e, bp0e = params['eprojs'][0]
    wp1e, bp1e = params['eprojs'][1]
    a1, c1, a2, c2 = _prep(we, _r1(be), wp0e, _r1(bp0e), wp1e, _r1(bp1e))

    wn, bn = params['node_enc']
    h = _matmul_rows(xp, wn, _r1(bn), BLK)

    edge_mats = [(we, _r1(be)), (a1, c1), (a2, c2)]
    widths = [256, 128, 64]
    for i in range(3):
        hw = widths[i]
        A, cvec = edge_mats[i]
        if hw > HC:
            plist = []
            for j in range(hw // HC):
                e_j = _matmul_rows(ea, A[:, j * HC:(j + 1) * HC],
                                   cvec[:, j * HC:(j + 1) * HC], EBLK)
                plist.append(_sc_msg(h[:, j * HC:(j + 1) * HC], e_j, src, dst))
            parts = jnp.concatenate(plist, axis=2)
        elif hw == HC:
            e_i = _matmul_rows(ea, A, cvec, EBLK)
            parts = _sc_msg(h, e_i, src, dst)
        else:
            ap = jnp.zeros((16, HC), F32).at[:, :hw].set(A)
            cp = jnp.zeros((1, HC), F32).at[:, :hw].set(cvec)
            e_i = _matmul_rows(ea, ap, cp, EBLK)
            xpad = jnp.zeros((N_PAD, HC), F32).at[:, :hw].set(h)
            parts = _sc_msg(xpad, e_i, src, dst)[:, :, :hw]
        w1, b1, w2, b2 = params['convs'][i]
        y, st = _layerA(h, parts, w1, _r1(b1), w2, _r1(b2))
        gamma, beta = params['bns'][i]
        if i < 2:
            wp, bp = params['projs'][i]
            h = _layerB(y, st, _r1(gamma), _r1(beta), wp, _r1(bp))
        else:
            h = _layerB(y, st, _r1(gamma), _r1(beta))

    pooled, cnts = _pool(h, batch_r)
    ew1, eb1, ew2, eb2 = params['ext']
    hw1, hb1, hw2, hb2 = params['head']
    out = _head(pooled, cnts, externals,
                ew1, _r1(eb1), ew2, _r1(eb2),
                hw1, _r1(hb1), _r1(hw2[:, 0]),
                jnp.broadcast_to(hb2.reshape(1, 1), (1, G)))
    return out[0]


def kernel(x, edge_attr, externals, params, edge_index, batch):
    return _run(x, edge_attr, externals, params, edge_index, batch)
